# baseline (speedup 1.0000x reference)
import jax
import jax.numpy as jnp
from jax.experimental import pallas as pl


def _add_block(x_ref, pe_ref, o_ref):
    o_ref[...] = x_ref[...] + pe_ref[...]


def kernel(x, pos_emb):
    b, t, c = x.shape
    bt = 512
    grid = (t // bt,)
    return pl.pallas_call(
        _add_block,
        grid=grid,
        in_specs=[
            pl.BlockSpec((b, bt, c), lambda i: (0, i, 0)),
            pl.BlockSpec((bt, c), lambda i: (i, 0)),
        ],
        out_specs=pl.BlockSpec((b, bt, c), lambda i: (0, i, 0)),
        out_shape=jax.ShapeDtypeStruct((b, t, c), x.dtype),
    )(x, pos_emb)


# speedup vs baseline: 1.0087x; 1.0087x over previous
import jax
import jax.numpy as jnp
from jax.experimental import pallas as pl


def _add_block(x_ref, pe_ref, o_ref):
    o_ref[...] = x_ref[...] + pe_ref[...]


def kernel(x, pos_emb):
    b, t, c = x.shape
    bc = 512
    grid = (c // bc, b)
    return pl.pallas_call(
        _add_block,
        grid=grid,
        in_specs=[
            pl.BlockSpec((1, t, bc), lambda i, j: (j, 0, i)),
            pl.BlockSpec((t, bc), lambda i, j: (0, i)),
        ],
        out_specs=pl.BlockSpec((1, t, bc), lambda i, j: (j, 0, i)),
        out_shape=jax.ShapeDtypeStruct((b, t, c), x.dtype),
    )(x, pos_emb)
